# baseline (speedup 1.0000x reference)
import jax
import jax.numpy as jnp
from jax import lax
from jax.experimental import pallas as pl
from jax.experimental.pallas import tpu as pltpu
from jax.experimental.pallas import tpu_sc as plsc

N_FEAT = 256
NWORD = N_FEAT // 2
L = 16
NC = 2
NS = 16
C = 80
N0 = 69
N1 = 56
E_PAD = NS * (N0 + N1) * C


NB = 5


def _dot_kernel(src_hbm, dst_hbm, ufeat_hbm, ifeat_hbm, out_hbm,
                sidx, didx, *bufs):
    cid = lax.axis_index("c")
    sid = lax.axis_index("s")
    row_ids = lax.iota(jnp.int32, L)
    ubuf = bufs[0:NB]
    vbuf = bufs[NB:2 * NB]
    obuf = bufs[2 * NB:3 * NB]
    usem = bufs[3 * NB:4 * NB]
    vsem = bufs[4 * NB:5 * NB]
    osem = bufs[5 * NB:6 * NB]

    def run_core(n_chunks, w_base):
        e_per_w = n_chunks * C
        pltpu.sync_copy(src_hbm.at[pl.ds(w_base, e_per_w)],
                        sidx.at[pl.ds(0, e_per_w)])
        pltpu.sync_copy(dst_hbm.at[pl.ds(w_base, e_per_w)],
                        didx.at[pl.ds(0, e_per_w)])

        def fire(i, b):
            off = i * C
            pltpu.async_copy(
                ufeat_hbm.at[sidx.at[pl.ds(off, C)]], ubuf[b], usem[b])
            pltpu.async_copy(
                ifeat_hbm.at[didx.at[pl.ds(off, C)]], vbuf[b], vsem[b])

        def wait_rows(b):
            pltpu.make_async_copy(
                ufeat_hbm.at[sidx.at[pl.ds(0, C)]], ubuf[b], usem[b]).wait()
            pltpu.make_async_copy(
                ifeat_hbm.at[didx.at[pl.ds(0, C)]], vbuf[b], vsem[b]).wait()

        def wait_out(b):
            pltpu.make_async_copy(
                obuf[b], out_hbm.at[pl.ds(0, C)], osem[b]).wait()

        def compute(i, b, guarded):
            wait_rows(b)
            if guarded:
                @pl.when(i + NB - 1 < n_chunks)
                def _():
                    fire(i + NB - 1, (b + NB - 1) % NB)

            @pl.when(i >= NB)
            def _():
                wait_out(b)

            urows = ubuf[b]
            vrows = vbuf[b]

            def group_body(g, _):
                base = g * L
                res = jnp.zeros((L,), jnp.float32)
                for e in range(L):
                    acc = jnp.zeros((L,), jnp.float32)
                    for j in range(NWORD // L):
                        uw = urows[base + e, pl.ds(j * L, L)]
                        vw = vrows[base + e, pl.ds(j * L, L)]
                        uhi = plsc.bitcast(uw, jnp.float32)
                        ulo = plsc.bitcast(uw << 16, jnp.float32)
                        vhi = plsc.bitcast(vw, jnp.float32)
                        vlo = plsc.bitcast(vw << 16, jnp.float32)
                        acc += uhi * vhi
                        acc += ulo * vlo
                    s = jnp.sum(acc)
                    res = jnp.where(row_ids == e, s, res)
                obuf[b][pl.ds(base, L)] = res
                return 0

            lax.fori_loop(0, C // L, group_body, 0)
            pltpu.async_copy(
                obuf[b], out_hbm.at[pl.ds(w_base + i * C, C)], osem[b])

        for j in range(NB - 1):
            fire(j, j)

        def outer_body(o, _):
            for b in range(NB):
                compute(o * NB + b, b, guarded=True)
            return 0

        lax.fori_loop(0, n_chunks // NB, outer_body, 0)
        for r in range(n_chunks % NB):
            i = (n_chunks // NB) * NB + r
            compute(i, i % NB, guarded=False)
        for b in range(NB):
            wait_out(b)

    @pl.when(cid == 0)
    def _():
        run_core(N0, sid * (N0 * C))

    @pl.when(cid == 1)
    def _():
        run_core(N1, NS * (N0 * C) + sid * (N1 * C))


def kernel(ufeat, ifeat, Q, edge_index):
    del Q
    e = edge_index.shape[1]
    assert e <= E_PAD
    src = edge_index[0].astype(jnp.int32)
    dst = edge_index[1].astype(jnp.int32)
    if E_PAD != e:
        src = jnp.pad(src, (0, E_PAD - e))
        dst = jnp.pad(dst, (0, E_PAD - e))

    def pack(x):
        xb = x.astype(jnp.bfloat16)
        hi = lax.bitcast_convert_type(xb[:, :NWORD], jnp.uint16)
        lo = lax.bitcast_convert_type(xb[:, NWORD:], jnp.uint16)
        packed = (hi.astype(jnp.uint32) << 16) | lo.astype(jnp.uint32)
        return lax.bitcast_convert_type(packed, jnp.int32)

    run = pl.kernel(
        _dot_kernel,
        out_type=jax.ShapeDtypeStruct((E_PAD,), jnp.float32),
        mesh=plsc.VectorSubcoreMesh(
            core_axis_name="c", subcore_axis_name="s",
            num_cores=NC, num_subcores=NS),
        scratch_types=[
            pltpu.VMEM((N0 * C,), jnp.int32),
            pltpu.VMEM((N0 * C,), jnp.int32),
            *[pltpu.VMEM((C, NWORD), jnp.int32) for _ in range(2 * NB)],
            *[pltpu.VMEM((C,), jnp.float32) for _ in range(NB)],
            *[pltpu.SemaphoreType.DMA for _ in range(3 * NB)],
        ],
        compiler_params=pltpu.CompilerParams(needs_layout_passes=False),
    )
    out = run(src, dst, pack(ufeat), pack(ifeat))
    return out[:e, None]


# speedup vs baseline: 1.0996x; 1.0996x over previous
import jax
import jax.numpy as jnp
from jax import lax
from jax.experimental import pallas as pl
from jax.experimental.pallas import tpu as pltpu
from jax.experimental.pallas import tpu_sc as plsc

N_FEAT = 256
NWORD = N_FEAT // 2
L = 16
NC = 2
NS = 16
C = 80
N0 = 64
N1 = 61
E_PAD = NS * (N0 + N1) * C


NB = 4


def _dot_kernel(src_hbm, dst_hbm, ufeat_hbm, ifeat_hbm, out_hbm,
                sidx, didx, *bufs):
    cid = lax.axis_index("c")
    sid = lax.axis_index("s")
    row_ids = lax.iota(jnp.int32, L)
    ubuf = bufs[0:NB]
    vbuf = bufs[NB:2 * NB]
    obuf = bufs[2 * NB:3 * NB]
    usem = bufs[3 * NB:4 * NB]
    vsem = bufs[4 * NB:5 * NB]
    osem = bufs[5 * NB:6 * NB]

    def run_core(n_chunks, w_base):
        e_per_w = n_chunks * C
        pltpu.sync_copy(src_hbm.at[pl.ds(w_base, e_per_w)],
                        sidx.at[pl.ds(0, e_per_w)])
        pltpu.sync_copy(dst_hbm.at[pl.ds(w_base, e_per_w)],
                        didx.at[pl.ds(0, e_per_w)])

        def fire(i, b):
            off = i * C
            pltpu.async_copy(
                ufeat_hbm.at[sidx.at[pl.ds(off, C)]], ubuf[b], usem[b])
            pltpu.async_copy(
                ifeat_hbm.at[didx.at[pl.ds(off, C)]], vbuf[b], vsem[b])

        def wait_rows(b):
            pltpu.make_async_copy(
                ufeat_hbm.at[sidx.at[pl.ds(0, C)]], ubuf[b], usem[b]).wait()
            pltpu.make_async_copy(
                ifeat_hbm.at[didx.at[pl.ds(0, C)]], vbuf[b], vsem[b]).wait()

        def wait_out(b):
            pltpu.make_async_copy(
                obuf[b], out_hbm.at[pl.ds(0, C)], osem[b]).wait()

        def compute(i, b, guarded):
            wait_rows(b)
            if guarded:
                @pl.when(i + NB - 1 < n_chunks)
                def _():
                    fire(i + NB - 1, (b + NB - 1) % NB)

            @pl.when(i >= NB)
            def _():
                wait_out(b)

            urows = ubuf[b]
            vrows = vbuf[b]

            def group_body(g, _):
                base = g * L
                res = jnp.zeros((L,), jnp.float32)
                for e in range(L):
                    acc = jnp.zeros((L,), jnp.float32)
                    for j in range(NWORD // L):
                        uw = urows[base + e, pl.ds(j * L, L)]
                        vw = vrows[base + e, pl.ds(j * L, L)]
                        uhi = plsc.bitcast(uw, jnp.float32)
                        ulo = plsc.bitcast(uw << 16, jnp.float32)
                        vhi = plsc.bitcast(vw, jnp.float32)
                        vlo = plsc.bitcast(vw << 16, jnp.float32)
                        acc += uhi * vhi
                        acc += ulo * vlo
                    s = jnp.sum(acc)
                    res = jnp.where(row_ids == e, s, res)
                obuf[b][pl.ds(base, L)] = res
                return 0

            lax.fori_loop(0, C // L, group_body, 0)
            pltpu.async_copy(
                obuf[b], out_hbm.at[pl.ds(w_base + i * C, C)], osem[b])

        for j in range(NB - 1):
            fire(j, j)

        def outer_body(o, _):
            for b in range(NB):
                compute(o * NB + b, b, guarded=True)
            return 0

        lax.fori_loop(0, n_chunks // NB, outer_body, 0)
        for r in range(n_chunks % NB):
            i = (n_chunks // NB) * NB + r
            compute(i, i % NB, guarded=False)
        for b in range(NB):
            wait_out(b)

    @pl.when(cid == 0)
    def _():
        run_core(N0, sid * (N0 * C))

    @pl.when(cid == 1)
    def _():
        run_core(N1, NS * (N0 * C) + sid * (N1 * C))


def kernel(ufeat, ifeat, Q, edge_index):
    del Q
    e = edge_index.shape[1]
    assert e <= E_PAD
    src = edge_index[0].astype(jnp.int32)
    dst = edge_index[1].astype(jnp.int32)
    if E_PAD != e:
        src = jnp.pad(src, (0, E_PAD - e))
        dst = jnp.pad(dst, (0, E_PAD - e))

    def pack(x):
        w = lax.bitcast_convert_type(x, jnp.int32) + 0x8000
        hi = w[:, :NWORD] & jnp.int32(-65536)
        lo = lax.shift_right_logical(w[:, NWORD:], 16)
        return hi | lo

    run = pl.kernel(
        _dot_kernel,
        out_type=jax.ShapeDtypeStruct((E_PAD,), jnp.float32),
        mesh=plsc.VectorSubcoreMesh(
            core_axis_name="c", subcore_axis_name="s",
            num_cores=NC, num_subcores=NS),
        scratch_types=[
            pltpu.VMEM((N0 * C,), jnp.int32),
            pltpu.VMEM((N0 * C,), jnp.int32),
            *[pltpu.VMEM((C, NWORD), jnp.int32) for _ in range(2 * NB)],
            *[pltpu.VMEM((C,), jnp.float32) for _ in range(NB)],
            *[pltpu.SemaphoreType.DMA for _ in range(3 * NB)],
        ],
        compiler_params=pltpu.CompilerParams(needs_layout_passes=False),
    )
    out = run(src, dst, pack(ufeat), pack(ifeat))
    return out[:e, None]
